# Initial kernel scaffold; baseline (speedup 1.0000x reference)
import functools

import jax
import jax.numpy as jnp
from jax import lax
from jax.experimental import pallas as pl
from jax.experimental.pallas import tpu as pltpu
from jax.experimental.pallas import tpu_sc as plsc

_LANES = 16
_N_SUBCORES = 16
_N_CORES = 2


def _degree_body(n_pad, n_stages, stage, unroll, edge_ref, out_ref, hist_v, idx_v):
    c = lax.axis_index("c")
    s = lax.axis_index("s")

    zeros16 = jnp.zeros((_LANES,), jnp.float32)
    ones16 = jnp.ones((_LANES,), jnp.float32)

    def zero_body(j, carry):
        for u in range(16):
            hist_v[pl.ds((j * 16 + u) * _LANES, _LANES)] = zeros16
        return carry
    lax.fori_loop(0, n_pad // (16 * _LANES), zero_body, 0)

    per_tile = n_stages * stage
    vregs_per_stage = stage // _LANES

    def stage_body(t, carry):
        base = s * per_tile + t * stage
        pltpu.sync_copy(edge_ref.at[c, pl.ds(base, stage)], idx_v)

        def vreg_body(j, inner):
            for u in range(unroll):
                k = j * unroll + u
                idx16 = idx_v[pl.ds(k * _LANES, _LANES)]
                plsc.addupdate_scatter(hist_v, [idx16], ones16)
            return inner
        lax.fori_loop(0, vregs_per_stage // unroll, vreg_body, 0)
        return carry

    lax.fori_loop(0, n_stages, stage_body, 0)

    pltpu.sync_copy(hist_v, out_ref.at[c, s])


def _count_degrees(edge_index, n_pad, stage=10000, unroll=5):
    n_edges = edge_index.shape[1]
    assert n_edges % (_N_SUBCORES * stage) == 0
    n_stages = n_edges // (_N_SUBCORES * stage)
    assert (stage // _LANES) % unroll == 0

    mesh = plsc.VectorSubcoreMesh(core_axis_name="c", subcore_axis_name="s")
    kfn = pl.kernel(
        functools.partial(_degree_body, n_pad, n_stages, stage, unroll),
        out_type=jax.ShapeDtypeStruct((_N_CORES, _N_SUBCORES, n_pad), jnp.float32),
        mesh=mesh,
        scratch_types=[
            pltpu.VMEM((n_pad,), jnp.float32),
            pltpu.VMEM((stage,), jnp.int32),
        ],
    )
    return kfn(edge_index)


def _apply_body(nbuckets, x_ref, deg_ref, iemb_ref, oemb_ref, o_ref):
    deg = jnp.sum(deg_ref[...], axis=1)
    iota = lax.broadcasted_iota(jnp.float32, (1, nbuckets), 1)

    def onehot(d):
        b = jnp.floor(jnp.log2(d + 1.0))
        b = jnp.clip(b, 0.0, float(nbuckets - 1))
        return (b == iota).astype(jnp.float32)

    acc = jnp.dot(onehot(deg[1]), iemb_ref[...],
                  preferred_element_type=jnp.float32)
    acc = acc + jnp.dot(onehot(deg[0]), oemb_ref[...],
                        preferred_element_type=jnp.float32)
    o_ref[...] = x_ref[...] + acc


def _apply_embeddings(x, deg_partials, in_emb, out_emb, block=2000):
    n, ch = x.shape
    nbuckets = in_emb.shape[0]
    assert n % block == 0
    grid = (n // block,)
    deg4 = deg_partials[..., None]
    return pl.pallas_call(
        functools.partial(_apply_body, nbuckets),
        grid=grid,
        in_specs=[
            pl.BlockSpec((block, ch), lambda i: (i, 0)),
            pl.BlockSpec((_N_CORES, _N_SUBCORES, block, 1), lambda i: (0, 0, i, 0)),
            pl.BlockSpec((nbuckets, ch), lambda i: (0, 0)),
            pl.BlockSpec((nbuckets, ch), lambda i: (0, 0)),
        ],
        out_specs=pl.BlockSpec((block, ch), lambda i: (i, 0)),
        out_shape=jax.ShapeDtypeStruct((n, ch), jnp.float32),
    )(x, deg4, in_emb, out_emb)


def kernel(x, edge_index, in_emb, out_emb):
    n = x.shape[0]
    n_pad = ((n + 255) // 256) * 256
    deg_partials = _count_degrees(edge_index, n_pad)
    return _apply_embeddings(x, deg_partials, in_emb, out_emb)


# speedup vs baseline: 17.0078x; 17.0078x over previous
import functools

import jax
import jax.numpy as jnp
from jax import lax
from jax.experimental import pallas as pl
from jax.experimental.pallas import tpu as pltpu
from jax.experimental.pallas import tpu_sc as plsc

_LANES = 16
_N_SUBCORES = 16
_N_CORES = 2
_REDUCE_ROUNDS = 8


def _degree_body(n_pad, n_stages, stage, unroll,
                 edge_ref, out_ref, hist_v, idx_v, shared_v, acc_v, tmp_v):
    c = lax.axis_index("c")
    s = lax.axis_index("s")

    zeros16 = jnp.zeros((_LANES,), jnp.float32)
    ones16 = jnp.ones((_LANES,), jnp.float32)

    def zero_body(j, carry):
        for u in range(16):
            hist_v[pl.ds((j * 16 + u) * _LANES, _LANES)] = zeros16
        return carry
    lax.fori_loop(0, n_pad // (16 * _LANES), zero_body, 0)

    per_tile = n_stages * stage
    vregs_per_stage = stage // _LANES

    def stage_body(t, carry):
        base = s * per_tile + t * stage
        pltpu.sync_copy(edge_ref.at[c, pl.ds(base, stage)], idx_v)

        def vreg_body(j, inner):
            for u in range(unroll):
                k = j * unroll + u
                idx16 = idx_v[pl.ds(k * _LANES, _LANES)]
                plsc.addupdate_scatter(hist_v, [idx16], ones16)
            return inner
        lax.fori_loop(0, vregs_per_stage // unroll, vreg_body, 0)
        return carry

    lax.fori_loop(0, n_stages, stage_body, 0)

    quarter = n_pad // _REDUCE_ROUNDS
    chunk = quarter // _N_SUBCORES
    n_vregs = chunk // _LANES
    red_unroll = 2 if n_vregs % 2 == 0 else 1
    for r in range(_REDUCE_ROUNDS):
        pltpu.sync_copy(hist_v.at[pl.ds(r * quarter, quarter)], shared_v.at[s])
        plsc.subcore_barrier()

        pltpu.sync_copy(shared_v.at[0, pl.ds(s * chunk, chunk)], acc_v)
        for k in range(1, _N_SUBCORES):
            pltpu.sync_copy(shared_v.at[k, pl.ds(s * chunk, chunk)], tmp_v)

            def add_body(j, carry):
                for u in range(red_unroll):
                    sl = pl.ds((j * red_unroll + u) * _LANES, _LANES)
                    acc_v[sl] = acc_v[sl] + tmp_v[sl]
                return carry
            lax.fori_loop(0, n_vregs // red_unroll, add_body, 0)

        pltpu.sync_copy(acc_v, out_ref.at[c, pl.ds(r * quarter + s * chunk, chunk)])
        plsc.subcore_barrier()


def _count_degrees(edge_index, n_pad, stage=4000, unroll=5):
    n_edges = edge_index.shape[1]
    assert n_edges % (_N_SUBCORES * stage) == 0
    n_stages = n_edges // (_N_SUBCORES * stage)
    assert (stage // _LANES) % unroll == 0
    quarter = n_pad // _REDUCE_ROUNDS
    chunk = quarter // _N_SUBCORES
    assert quarter * _REDUCE_ROUNDS == n_pad
    assert chunk * _N_SUBCORES == quarter
    assert chunk % _LANES == 0 and chunk % 8 == 0

    mesh = plsc.VectorSubcoreMesh(
        core_axis_name="c", subcore_axis_name="s",
        num_cores=_N_CORES, num_subcores=_N_SUBCORES)
    kfn = pl.kernel(
        functools.partial(_degree_body, n_pad, n_stages, stage, unroll),
        out_type=jax.ShapeDtypeStruct((_N_CORES, n_pad), jnp.float32),
        mesh=mesh,
        scratch_types=[
            pltpu.VMEM((n_pad,), jnp.float32),
            pltpu.VMEM((stage,), jnp.int32),
            pltpu.VMEM_SHARED((_N_SUBCORES, quarter), jnp.float32),
            pltpu.VMEM((chunk,), jnp.float32),
            pltpu.VMEM((chunk,), jnp.float32),
        ],
        compiler_params=pltpu.CompilerParams(
            use_tc_tiling_on_sc=False, needs_layout_passes=False),
    )
    return kfn(edge_index)


def _apply_body(nbuckets, x_ref, deg_ref, iemb_ref, oemb_ref, o_ref):
    deg = deg_ref[...]
    iota = lax.broadcasted_iota(jnp.int32, (1, nbuckets), 1)

    def onehot(d):
        b = jnp.floor(jnp.log2(d + 1.0))
        b = jnp.clip(b, 0, nbuckets - 1).astype(jnp.int32)
        return (b == iota).astype(jnp.float32)

    acc = jnp.dot(onehot(deg[1]), iemb_ref[...],
                  preferred_element_type=jnp.float32)
    acc = acc + jnp.dot(onehot(deg[0]), oemb_ref[...],
                        preferred_element_type=jnp.float32)
    o_ref[...] = x_ref[...] + acc


def _apply_embeddings(x, deg_partials, in_emb, out_emb, block=2000):
    n, ch = x.shape
    nbuckets = in_emb.shape[0]
    assert n % block == 0
    grid = (n // block,)
    deg4 = deg_partials[..., None]
    return pl.pallas_call(
        functools.partial(_apply_body, nbuckets),
        grid=grid,
        in_specs=[
            pl.BlockSpec((block, ch), lambda i: (i, 0)),
            pl.BlockSpec((_N_CORES, block, 1), lambda i: (0, i, 0)),
            pl.BlockSpec((nbuckets, ch), lambda i: (0, 0)),
            pl.BlockSpec((nbuckets, ch), lambda i: (0, 0)),
        ],
        out_specs=pl.BlockSpec((block, ch), lambda i: (i, 0)),
        out_shape=jax.ShapeDtypeStruct((n, ch), jnp.float32),
    )(x, deg4, in_emb, out_emb)


def kernel(x, edge_index, in_emb, out_emb):
    n = x.shape[0]
    n_pad = ((n + 1023) // 1024) * 1024
    deg_partials = _count_degrees(edge_index, n_pad)
    return _apply_embeddings(x, deg_partials, in_emb, out_emb)


# speedup vs baseline: 24.5609x; 1.4441x over previous
import functools

import jax
import jax.numpy as jnp
from jax import lax
from jax.experimental import pallas as pl
from jax.experimental.pallas import tpu as pltpu
from jax.experimental.pallas import tpu_sc as plsc

_LANES = 16
_N_SUBCORES = 16
_N_CORES = 2
_REDUCE_ROUNDS = 8


def _degree_body(n_pad, n_stages, stage, unroll,
                 edge_ref, out_ref, hist_v, idx_v, shared_v, acc_v, tmp_v):
    c = lax.axis_index("c")
    s = lax.axis_index("s")

    zeros16 = jnp.zeros((_LANES,), jnp.float32)
    ones16 = jnp.ones((_LANES,), jnp.float32)

    def zero_body(j, carry):
        for u in range(16):
            hist_v[pl.ds((j * 16 + u) * _LANES, _LANES)] = zeros16
        return carry
    lax.fori_loop(0, n_pad // (16 * _LANES), zero_body, 0)

    per_tile = n_stages * stage
    vregs_per_stage = stage // _LANES

    def stage_body(t, carry):
        base = s * per_tile + t * stage
        pltpu.sync_copy(edge_ref.at[c, pl.ds(base, stage)], idx_v)

        def vreg_body(j, inner):
            for u in range(unroll):
                k = j * unroll + u
                idx16 = idx_v[pl.ds(k * _LANES, _LANES)]
                plsc.addupdate_scatter(hist_v, [idx16], ones16)
            return inner
        lax.fori_loop(0, vregs_per_stage // unroll, vreg_body, 0)
        return carry

    lax.fori_loop(0, n_stages, stage_body, 0)

    quarter = n_pad // _REDUCE_ROUNDS
    chunk = quarter // _N_SUBCORES
    n_vregs = chunk // _LANES
    red_unroll = 2 if n_vregs % 2 == 0 else 1
    for r in range(_REDUCE_ROUNDS):
        pltpu.sync_copy(hist_v.at[pl.ds(r * quarter, quarter)], shared_v.at[s])
        plsc.subcore_barrier()

        pltpu.sync_copy(shared_v.at[0, pl.ds(s * chunk, chunk)], acc_v)
        for k in range(1, _N_SUBCORES):
            pltpu.sync_copy(shared_v.at[k, pl.ds(s * chunk, chunk)], tmp_v)

            def add_body(j, carry):
                for u in range(red_unroll):
                    sl = pl.ds((j * red_unroll + u) * _LANES, _LANES)
                    acc_v[sl] = acc_v[sl] + tmp_v[sl]
                return carry
            lax.fori_loop(0, n_vregs // red_unroll, add_body, 0)

        pltpu.sync_copy(acc_v, out_ref.at[c, pl.ds(r * quarter + s * chunk, chunk)])
        plsc.subcore_barrier()


def _count_degrees(edge_index, n_pad, stage=4000, unroll=5):
    n_edges = edge_index.shape[1]
    assert n_edges % (_N_SUBCORES * stage) == 0
    n_stages = n_edges // (_N_SUBCORES * stage)
    assert (stage // _LANES) % unroll == 0
    quarter = n_pad // _REDUCE_ROUNDS
    chunk = quarter // _N_SUBCORES
    assert quarter * _REDUCE_ROUNDS == n_pad
    assert chunk * _N_SUBCORES == quarter
    assert chunk % _LANES == 0 and chunk % 8 == 0

    mesh = plsc.VectorSubcoreMesh(
        core_axis_name="c", subcore_axis_name="s",
        num_cores=_N_CORES, num_subcores=_N_SUBCORES)
    kfn = pl.kernel(
        functools.partial(_degree_body, n_pad, n_stages, stage, unroll),
        out_type=jax.ShapeDtypeStruct((_N_CORES, n_pad), jnp.float32),
        mesh=mesh,
        scratch_types=[
            pltpu.VMEM((n_pad,), jnp.float32),
            pltpu.VMEM((stage,), jnp.int32),
            pltpu.VMEM_SHARED((_N_SUBCORES, quarter), jnp.float32),
            pltpu.VMEM((chunk,), jnp.float32),
            pltpu.VMEM((chunk,), jnp.float32),
        ],
        compiler_params=pltpu.CompilerParams(
            use_tc_tiling_on_sc=False, needs_layout_passes=False),
    )
    return kfn(edge_index)


def _apply_body(nbuckets, x_ref, deg_ref, iemb_ref, oemb_ref, o_ref):
    deg = deg_ref[...]
    iota = lax.broadcasted_iota(jnp.int32, (nbuckets, 1), 0)

    def onehot_t(d):
        b = jnp.floor(jnp.log2(d + 1.0))
        b = jnp.clip(b, 0, nbuckets - 1).astype(jnp.int32)
        return (b == iota).astype(jnp.float32)

    contract = (((0,), (0,)), ((), ()))
    acc = lax.dot_general(onehot_t(deg[1:2]), iemb_ref[...], contract,
                          preferred_element_type=jnp.float32)
    acc = acc + lax.dot_general(onehot_t(deg[0:1]), oemb_ref[...], contract,
                                preferred_element_type=jnp.float32)
    o_ref[...] = x_ref[...] + acc


def _apply_embeddings(x, deg_partials, in_emb, out_emb, block=2048):
    n, ch = x.shape
    nbuckets = in_emb.shape[0]
    grid = ((n + block - 1) // block,)
    return pl.pallas_call(
        functools.partial(_apply_body, nbuckets),
        grid=grid,
        in_specs=[
            pl.BlockSpec((block, ch), lambda i: (i, 0)),
            pl.BlockSpec((_N_CORES, block), lambda i: (0, i)),
            pl.BlockSpec((nbuckets, ch), lambda i: (0, 0)),
            pl.BlockSpec((nbuckets, ch), lambda i: (0, 0)),
        ],
        out_specs=pl.BlockSpec((block, ch), lambda i: (i, 0)),
        out_shape=jax.ShapeDtypeStruct((n, ch), jnp.float32),
    )(x, deg_partials, in_emb, out_emb)


def kernel(x, edge_index, in_emb, out_emb):
    n = x.shape[0]
    n_pad = ((n + 1023) // 1024) * 1024
    deg_partials = _count_degrees(edge_index, n_pad)
    return _apply_embeddings(x, deg_partials, in_emb, out_emb)


# speedup vs baseline: 33.8281x; 1.3773x over previous
import functools

import jax
import jax.numpy as jnp
from jax import lax
from jax.experimental import pallas as pl
from jax.experimental.pallas import tpu as pltpu
from jax.experimental.pallas import tpu_sc as plsc

_LANES = 16
_N_SUBCORES = 16
_N_CORES = 2
_REDUCE_ROUNDS = 8


def _degree_body(n_pad, n_stages, stage, unroll,
                 edge_ref, out_ref, hist_v, idx_a, idx_b, shared_v, acc_v,
                 tmp_v, sem_a, sem_b):
    c = lax.axis_index("c")
    s = lax.axis_index("s")

    zeros16 = jnp.zeros((_LANES,), jnp.float32)
    ones16 = jnp.ones((_LANES,), jnp.float32)

    per_tile = n_stages * stage
    vregs_per_stage = stage // _LANES

    def edge_slice(t):
        return edge_ref.at[c, pl.ds(s * per_tile + t * stage, stage)]

    pltpu.async_copy(edge_slice(0), idx_a, sem_a)
    pltpu.async_copy(edge_slice(1), idx_b, sem_b)

    def zero_body(j, carry):
        for u in range(16):
            hist_v[pl.ds((j * 16 + u) * _LANES, _LANES)] = zeros16
        return carry
    lax.fori_loop(0, n_pad // (16 * _LANES), zero_body, 0)

    def process(buf):
        def vreg_body(j, inner):
            for u in range(unroll):
                k = j * unroll + u
                idx16 = buf[pl.ds(k * _LANES, _LANES)]
                plsc.addupdate_scatter(hist_v, [idx16], ones16)
            return inner
        lax.fori_loop(0, vregs_per_stage // unroll, vreg_body, 0)

    def outer_body(i, carry):
        t2 = i * 2
        for b, (buf, sem) in enumerate(((idx_a, sem_a), (idx_b, sem_b))):
            pltpu.make_async_copy(edge_slice(0), buf, sem).wait()
            process(buf)
            pltpu.async_copy(edge_slice(t2 + b + 2), buf, sem)
        return carry
    lax.fori_loop(0, n_stages // 2 - 1, outer_body, 0)
    for buf, sem in ((idx_a, sem_a), (idx_b, sem_b)):
        pltpu.make_async_copy(edge_slice(0), buf, sem).wait()
        process(buf)

    slab = n_pad // _REDUCE_ROUNDS
    chunk = slab // _N_SUBCORES
    for r in range(_REDUCE_ROUNDS):
        pltpu.sync_copy(hist_v.at[pl.ds(r * slab, slab)], shared_v.at[s])
        plsc.subcore_barrier()

        pltpu.sync_copy(shared_v.at[:, pl.ds(s * chunk, chunk)], tmp_v)

        def red_body(j, carry):
            sl = pl.ds(j * _LANES, _LANES)
            v = tmp_v[0, sl]
            for k in range(1, _N_SUBCORES):
                v = v + tmp_v[k, sl]
            acc_v[sl] = v
            return carry
        lax.fori_loop(0, chunk // _LANES, red_body, 0)

        pltpu.sync_copy(acc_v, out_ref.at[c, pl.ds(r * slab + s * chunk, chunk)])
        plsc.subcore_barrier()


def _count_degrees(edge_index, n_pad, stage=2000, unroll=5):
    n_edges = edge_index.shape[1]
    assert n_edges % (_N_SUBCORES * stage) == 0
    n_stages = n_edges // (_N_SUBCORES * stage)
    assert n_stages % 2 == 0 and n_stages >= 4
    assert (stage // _LANES) % unroll == 0
    slab = n_pad // _REDUCE_ROUNDS
    chunk = slab // _N_SUBCORES
    assert slab * _REDUCE_ROUNDS == n_pad
    assert chunk * _N_SUBCORES == slab
    assert chunk % _LANES == 0 and chunk % 8 == 0

    mesh = plsc.VectorSubcoreMesh(
        core_axis_name="c", subcore_axis_name="s",
        num_cores=_N_CORES, num_subcores=_N_SUBCORES)
    kfn = pl.kernel(
        functools.partial(_degree_body, n_pad, n_stages, stage, unroll),
        out_type=jax.ShapeDtypeStruct((_N_CORES, n_pad), jnp.float32),
        mesh=mesh,
        scratch_types=[
            pltpu.VMEM((n_pad,), jnp.float32),
            pltpu.VMEM((stage,), jnp.int32),
            pltpu.VMEM((stage,), jnp.int32),
            pltpu.VMEM_SHARED((_N_SUBCORES, slab), jnp.float32),
            pltpu.VMEM((chunk,), jnp.float32),
            pltpu.VMEM((_N_SUBCORES, chunk), jnp.float32),
            pltpu.SemaphoreType.DMA,
            pltpu.SemaphoreType.DMA,
        ],
        compiler_params=pltpu.CompilerParams(
            use_tc_tiling_on_sc=False, needs_layout_passes=False),
    )
    return kfn(edge_index)


def _apply_body(nbuckets, x_ref, deg_ref, iemb_ref, oemb_ref, o_ref):
    deg = deg_ref[...]
    iota = lax.broadcasted_iota(jnp.int32, (nbuckets, 1), 0)

    def onehot_t(d):
        b = jnp.floor(jnp.log2(d + 1.0))
        b = jnp.clip(b, 0, nbuckets - 1).astype(jnp.int32)
        return (b == iota).astype(jnp.float32)

    contract = (((0,), (0,)), ((), ()))
    acc = lax.dot_general(onehot_t(deg[1:2]), iemb_ref[...], contract,
                          preferred_element_type=jnp.float32)
    acc = acc + lax.dot_general(onehot_t(deg[0:1]), oemb_ref[...], contract,
                                preferred_element_type=jnp.float32)
    o_ref[...] = x_ref[...] + acc


def _apply_embeddings(x, deg_partials, in_emb, out_emb, block=2048):
    n, ch = x.shape
    nbuckets = in_emb.shape[0]
    grid = ((n + block - 1) // block,)
    return pl.pallas_call(
        functools.partial(_apply_body, nbuckets),
        grid=grid,
        in_specs=[
            pl.BlockSpec((block, ch), lambda i: (i, 0)),
            pl.BlockSpec((_N_CORES, block), lambda i: (0, i)),
            pl.BlockSpec((nbuckets, ch), lambda i: (0, 0)),
            pl.BlockSpec((nbuckets, ch), lambda i: (0, 0)),
        ],
        out_specs=pl.BlockSpec((block, ch), lambda i: (i, 0)),
        out_shape=jax.ShapeDtypeStruct((n, ch), jnp.float32),
    )(x, deg_partials, in_emb, out_emb)


def kernel(x, edge_index, in_emb, out_emb):
    n = x.shape[0]
    n_pad = ((n + 1023) // 1024) * 1024
    deg_partials = _count_degrees(edge_index, n_pad)
    return _apply_embeddings(x, deg_partials, in_emb, out_emb)
